# Initial kernel scaffold; baseline (speedup 1.0000x reference)
import jax
import jax.numpy as jnp
from jax.experimental import pallas as pl


def kernel(encoded_doc, sentence_map, span_width_emb, span_width_prior_emb, W1, b1, W2, b2, Ww1, bw1, Ww2, bw2):
    raise NotImplementedError("write your pallas kernel here")


# speedup vs baseline: 10.3252x; 10.3252x over previous
import jax
import jax.numpy as jnp
from jax.experimental import pallas as pl
from jax.experimental.pallas import tpu as pltpu

_NW = 2048
_MW = 20
_HS = 768
_MLP = 256
_K = 819
_IN = 2 * _HS + _MW
_KP = 1792
_PAD = 32


def _bdot(a, b):
    return jax.lax.dot_general(a.astype(jnp.bfloat16), b.astype(jnp.bfloat16),
                               (((1,), (0,)), ((), ())),
                               preferred_element_type=jnp.float32)


def _logits_body(enc_ref, w1p_ref, b1_ref, w2_ref, b2_ref, wemb_ref,
                 wprior_ref, ww1p_ref, bw1_ref, ww2_ref, bw2_ref, sm_ref,
                 out_ref, x_ref, eext_ref):
    enc_bf = enc_ref[...].astype(jnp.bfloat16)
    eext_ref[pl.ds(0, _NW), :] = enc_bf
    eext_ref[pl.ds(_NW, _PAD), :] = jnp.broadcast_to(enc_bf[_NW - 1:_NW, :],
                                                     (_PAD, _HS))
    x_ref[:, 0:_HS] = enc_bf
    x_ref[:, _IN:_KP] = jnp.zeros((_NW, _KP - _IN), jnp.bfloat16)
    w1p = w1p_ref[...].astype(jnp.bfloat16)
    w2 = w2_ref[...].astype(jnp.bfloat16)
    wemb_bf = wemb_ref[...].astype(jnp.bfloat16)

    WH = jnp.maximum(_bdot(wprior_ref[...], ww1p_ref[...]) + bw1_ref[...], 0.0)
    WS = _bdot(WH, ww2_ref[...]) + bw2_ref[...]

    s_iota = jax.lax.broadcasted_iota(jnp.int32, (_NW, 1), 0)
    sm0 = sm_ref[pl.ds(0, _NW), :]
    b1 = b1_ref[...]
    neg_inf = jnp.float32(-jnp.inf)
    for w in range(_MW):
        x_ref[:, _HS:2 * _HS] = eext_ref[pl.ds(w, _NW), :]
        x_ref[:, 2 * _HS:_IN] = jnp.broadcast_to(wemb_bf[w:w + 1, :], (_NW, _MW))
        H = jnp.maximum(
            jax.lax.dot_general(x_ref[...], w1p,
                                (((1,), (0,)), ((), ())),
                                preferred_element_type=jnp.float32) + b1, 0.0)
        col = (jax.lax.dot_general(H.astype(jnp.bfloat16), w2,
                                   (((1,), (0,)), ((), ())),
                                   preferred_element_type=jnp.float32)
               + b2_ref[...]) + WS[w:w + 1, :]
        smw = sm_ref[pl.ds(w, _NW), :]
        valid = (s_iota < _NW - w) & (sm0 == smw)
        out_ref[:, w:w + 1] = jnp.where(valid, col, neg_inf)


def kernel(encoded_doc, sentence_map, span_width_emb, span_width_prior_emb,
           W1, b1, W2, b2, Ww1, bw1, Ww2, bw2):
    W1p = jnp.pad(W1, ((0, _KP - _IN), (0, 0)))
    prior_p = jnp.pad(span_width_prior_emb, ((0, 0), (0, _MLP - _MW)))
    Ww1p = jnp.pad(Ww1, ((0, _MLP - _MW), (0, 0)))
    sm = sentence_map.astype(jnp.int32)
    sm_ext = jnp.concatenate(
        [sm, jnp.broadcast_to(sm[-1], (_PAD,))]).reshape(_NW + _PAD, 1)

    logits = pl.pallas_call(
        _logits_body,
        out_shape=jax.ShapeDtypeStruct((_NW, _MW), jnp.float32),
        scratch_shapes=[pltpu.VMEM((_NW, _KP), jnp.bfloat16),
                        pltpu.VMEM((_NW + _PAD, _HS), jnp.bfloat16)],
    )(encoded_doc, W1p, b1.reshape(1, _MLP), W2, b2.reshape(1, 1),
      span_width_emb, prior_p, Ww1p, bw1.reshape(1, _MLP),
      Ww2, bw2.reshape(1, 1), sm_ext)

    flat = logits.reshape(-1)
    _, topk_idx = jax.lax.top_k(flat, _K)
    starts = topk_idx // _MW
    ends = starts + topk_idx % _MW
    scores = flat[topk_idx]
    order = jnp.argsort(starts.astype(jnp.float32)
                        + 1e-5 * ends.astype(jnp.float32))
    return starts[order], ends[order], scores[order]
